# baseline (speedup 1.0000x reference)
import functools

import jax
import jax.numpy as jnp
from jax import lax
from jax.experimental import pallas as pl
from jax.experimental.pallas import tpu as pltpu
from jax.experimental.pallas import tpu_sc as plsc

EMBED = 64
BATCH = 16384
LABELS = 50

NC, NS = 2, 16
NW = NC * NS
BB = 128
NB = BATCH // BB
BPW = NB // NW
LANES = 16
NVR = EMBED // LANES


def _sc_embed_sum(weight, idx):
    mesh = plsc.VectorSubcoreMesh(core_axis_name="c", subcore_axis_name="s")

    @functools.partial(
        pl.kernel,
        out_type=jax.ShapeDtypeStruct((BATCH, EMBED), jnp.float32),
        mesh=mesh,
        compiler_params=pltpu.CompilerParams(use_tc_tiling_on_sc=False,
                                              needs_layout_passes=False),
        scratch_types=[
            pltpu.VMEM((BB, LABELS), jnp.int32),
            pltpu.VMEM((BB,), jnp.int32),
            pltpu.VMEM((BB,), jnp.int32),
            pltpu.VMEM((BB, EMBED), jnp.float32),
            pltpu.VMEM((BB, EMBED), jnp.float32),
            pltpu.VMEM((BB, EMBED), jnp.float32),
            pltpu.SemaphoreType.DMA,
            pltpu.SemaphoreType.DMA,
        ],
    )
    def k(w_hbm, idx_hbm, out_hbm, idx_v, col0, col1, acc, buf0, buf1,
          sem0, sem1):
        wid = lax.axis_index("s") * NC + lax.axis_index("c")
        bufs = (buf0, buf1)
        cols = (col0, col1)
        sems = (sem0, sem1)
        zero = jnp.zeros((LANES,), jnp.float32)
        lane_iota = lax.iota(jnp.int32, LANES)

        def build_col(l, b):
            li = jnp.full((LANES,), l, jnp.int32)
            for c in range(BB // LANES):
                vals = plsc.load_gather(idx_v, [lane_iota + (c * LANES), li])
                cols[b][pl.ds(c * LANES, LANES)] = vals

        def gather_start(b):
            pltpu.make_async_copy(w_hbm.at[cols[b]], bufs[b], sems[b]).start()

        def gather_wait(b):
            pltpu.make_async_copy(w_hbm.at[cols[b]], bufs[b], sems[b]).wait()

        def accum(b):
            buf = bufs[b]

            def rbody(i, _):
                r0 = i * 4
                for kk in range(4):
                    for c in range(NVR):
                        sl = pl.ds(c * LANES, LANES)
                        plsc.addupdate(acc.at[r0 + kk, sl], buf[r0 + kk, sl])
                return 0

            lax.fori_loop(0, BB // 4, rbody, 0)

        def block_body(kblk, _):
            jb = wid * BPW + kblk
            pltpu.sync_copy(idx_hbm.at[pl.ds(jb * BB, BB)], idx_v)

            def zbody(i, _):
                r0 = i * 4
                for kk in range(4):
                    for c in range(NVR):
                        acc[r0 + kk, pl.ds(c * LANES, LANES)] = zero
                return 0

            lax.fori_loop(0, BB // 4, zbody, 0)

            build_col(0, 0)
            gather_start(0)
            build_col(1, 1)
            gather_start(1)

            def pair_body(i, _):
                for b in range(2):
                    l = 2 * i + b
                    gather_wait(b)
                    accum(b)
                    build_col(l + 2, b)
                    gather_start(b)
                return 0

            lax.fori_loop(0, LABELS // 2 - 1, pair_body, 0)

            for b in range(2):
                gather_wait(b)
                accum(b)

            pltpu.sync_copy(acc, out_hbm.at[pl.ds(jb * BB, BB)])
            return 0

        lax.fori_loop(0, BPW, block_body, 0)

    return k(weight, idx)


def kernel(inputs, weight):
    return _sc_embed_sum(weight, inputs.astype(jnp.int32))
